# baseline (speedup 1.0000x reference)
import functools
import jax
import jax.numpy as jnp
from jax import lax
from jax.experimental import pallas as pl
from jax.experimental.pallas import tpu as pltpu
from jax.experimental.pallas import tpu_sc as plsc

EMBED_DIM = 768
BATCH = 1024
SEQ = 77
SEQ_PAD = 80
NUM_WORKERS = 32
NB_PER_W = BATCH // NUM_WORKERS
TC_BB = 8


def _sc_gather_padded(table, idx):
  mesh = plsc.VectorSubcoreMesh(core_axis_name="c", subcore_axis_name="s")

  @functools.partial(
      pl.kernel,
      mesh=mesh,
      out_type=jax.ShapeDtypeStruct((BATCH, SEQ_PAD, EMBED_DIM), jnp.float32),
      scratch_types=[
          pltpu.VMEM((NB_PER_W, SEQ_PAD), jnp.int32),
          pltpu.VMEM((SEQ_PAD, EMBED_DIM), jnp.float32),
          pltpu.VMEM((SEQ_PAD, EMBED_DIM), jnp.float32),
          pltpu.SemaphoreType.DMA,
          pltpu.SemaphoreType.DMA,
          pltpu.SemaphoreType.DMA,
          pltpu.SemaphoreType.DMA,
      ],
  )
  def k(table_hbm, idx_hbm, out_hbm, idx_v, buf0, buf1,
        gsem0, gsem1, ssem0, ssem1):
    wid = lax.axis_index("s") * 2 + lax.axis_index("c")
    base = wid * NB_PER_W
    pltpu.sync_copy(idx_hbm.at[pl.ds(base, NB_PER_W)], idx_v)

    bufs = (buf0, buf1)
    gsems = (gsem0, gsem1)
    ssems = (ssem0, ssem1)

    def gather(j, p):
      return pltpu.async_copy(table_hbm.at[idx_v.at[j]], bufs[p], gsems[p])

    def store(j, p):
      return pltpu.async_copy(bufs[p], out_hbm.at[base + j], ssems[p])

    gather(0, 0).wait()
    gather(1, 1)
    store(0, 0)

    def pair_body(m, carry):
      g = 1 + 2 * m
      for (j, p) in ((g, 1), (g + 1, 0)):
        q = 1 - p
        pltpu.make_async_copy(
            table_hbm.at[idx_v.at[j]], bufs[p], gsems[p]).wait()
        pltpu.make_async_copy(
            bufs[q], out_hbm.at[base + j - 1], ssems[q]).wait()
        gather(j + 1, q)
        store(j, p)
      return carry

    lax.fori_loop(0, (NB_PER_W - 2) // 2, pair_body, 0, unroll=False)

    j_last = NB_PER_W - 1
    pltpu.make_async_copy(
        table_hbm.at[idx_v.at[j_last]], bufs[1], gsems[1]).wait()
    pltpu.make_async_copy(
        bufs[0], out_hbm.at[base + j_last - 1], ssems[0]).wait()
    store(j_last, 1)
    pltpu.make_async_copy(
        bufs[1], out_hbm.at[base + j_last], ssems[1]).wait()

  return k(table, idx)


def _tc_unpad(padded):
  def body(in_ref, out_ref):
    out_ref[...] = in_ref[:, :SEQ, :]

  return pl.pallas_call(
      body,
      grid=(BATCH // TC_BB,),
      in_specs=[pl.BlockSpec((TC_BB, SEQ_PAD, EMBED_DIM), lambda i: (i, 0, 0))],
      out_specs=pl.BlockSpec((TC_BB, SEQ, EMBED_DIM), lambda i: (i, 0, 0)),
      out_shape=jax.ShapeDtypeStruct((BATCH, SEQ, EMBED_DIM), jnp.float32),
  )(padded)


def kernel(x, token_embedding):
  idx_pad = jnp.pad(x.astype(jnp.int32), ((0, 0), (0, SEQ_PAD - SEQ)))
  padded = _sc_gather_padded(token_embedding, idx_pad)
  return _tc_unpad(padded)


# speedup vs baseline: 1.4449x; 1.4449x over previous
import functools
import jax
import jax.numpy as jnp
from jax import lax
from jax.experimental import pallas as pl
from jax.experimental.pallas import tpu as pltpu
from jax.experimental.pallas import tpu_sc as plsc

EMBED_DIM = 768
BATCH = 1024
SEQ = 77
SEQ_PAD = 80
NUM_WORKERS = 32
NB_PER_W = BATCH // NUM_WORKERS
TC_BB = 8


def _sc_gather_padded(table, idx):
  mesh = plsc.VectorSubcoreMesh(core_axis_name="c", subcore_axis_name="s")

  @functools.partial(
      pl.kernel,
      mesh=mesh,
      out_type=jax.ShapeDtypeStruct((BATCH, SEQ_PAD, EMBED_DIM), jnp.float32),
      scratch_types=[
          pltpu.VMEM((NB_PER_W, SEQ_PAD), jnp.int32),
          pltpu.VMEM((SEQ_PAD, EMBED_DIM), jnp.float32),
          pltpu.VMEM((SEQ_PAD, EMBED_DIM), jnp.float32),
          pltpu.SemaphoreType.DMA,
          pltpu.SemaphoreType.DMA,
          pltpu.SemaphoreType.DMA,
          pltpu.SemaphoreType.DMA,
      ],
  )
  def k(table_hbm, idx_hbm, out_hbm, idx_v, buf0, buf1,
        gsem0, gsem1, ssem0, ssem1):
    wid = lax.axis_index("s") * 2 + lax.axis_index("c")
    base = wid * NB_PER_W
    pltpu.sync_copy(idx_hbm.at[pl.ds(base, NB_PER_W)], idx_v)

    bufs = (buf0, buf1)
    gsems = (gsem0, gsem1)
    ssems = (ssem0, ssem1)

    def gather(j, p):
      return pltpu.async_copy(table_hbm.at[idx_v.at[j]], bufs[p], gsems[p])

    def store(j, p):
      return pltpu.async_copy(bufs[p], out_hbm.at[base + j], ssems[p])

    gather(0, 0).wait()
    gather(1, 1)
    store(0, 0)

    def pair_body(m, carry):
      g = 1 + 2 * m
      for (j, p) in ((g, 1), (g + 1, 0)):
        q = 1 - p
        pltpu.make_async_copy(
            table_hbm.at[idx_v.at[j]], bufs[p], gsems[p]).wait()
        pltpu.make_async_copy(
            bufs[q], out_hbm.at[base + j - 1], ssems[q]).wait()
        gather(j + 1, q)
        store(j, p)
      return carry

    lax.fori_loop(0, (NB_PER_W - 2) // 2, pair_body, 0, unroll=False)

    j_last = NB_PER_W - 1
    pltpu.make_async_copy(
        table_hbm.at[idx_v.at[j_last]], bufs[1], gsems[1]).wait()
    pltpu.make_async_copy(
        bufs[0], out_hbm.at[base + j_last - 1], ssems[0]).wait()
    store(j_last, 1)
    pltpu.make_async_copy(
        bufs[1], out_hbm.at[base + j_last], ssems[1]).wait()

  return k(table, idx)


def _tc_unpad(padded):
  def body(in_ref, out_ref):
    out_ref[...] = in_ref[:, :SEQ, :]

  return pl.pallas_call(
      body,
      grid=(BATCH // TC_BB,),
      in_specs=[pl.BlockSpec((TC_BB, SEQ_PAD, EMBED_DIM), lambda i: (i, 0, 0))],
      out_specs=pl.BlockSpec((TC_BB, SEQ, EMBED_DIM), lambda i: (i, 0, 0)),
      out_shape=jax.ShapeDtypeStruct((BATCH, SEQ, EMBED_DIM), jnp.float32),
  )(padded)


def kernel(x, token_embedding):
  idx_pad = jnp.pad(x.astype(jnp.int32), ((0, 0), (0, SEQ_PAD - SEQ)),
                    mode="edge")
  padded = _sc_gather_padded(token_embedding, idx_pad)
  return _tc_unpad(padded)


# speedup vs baseline: 2.0384x; 1.4108x over previous
import functools
import jax
import jax.numpy as jnp
from jax import lax
from jax.experimental import pallas as pl
from jax.experimental.pallas import tpu as pltpu
from jax.experimental.pallas import tpu_sc as plsc

EMBED_DIM = 768
BATCH = 1024
SEQ = 77
SEQ_PAD = 80
NUM_WORKERS = 32
NB_PER_W = BATCH // NUM_WORKERS
TC_BB = 8


def _sc_gather_padded(table, idx):
  mesh = plsc.VectorSubcoreMesh(core_axis_name="c", subcore_axis_name="s")

  @functools.partial(
      pl.kernel,
      mesh=mesh,
      out_type=jax.ShapeDtypeStruct((BATCH, SEQ_PAD, EMBED_DIM), jnp.float32),
      scratch_types=[
          pltpu.VMEM((NB_PER_W, SEQ_PAD), jnp.int32),
          pltpu.VMEM((SEQ_PAD, EMBED_DIM), jnp.float32),
          pltpu.VMEM((SEQ_PAD, EMBED_DIM), jnp.float32),
          pltpu.SemaphoreType.DMA,
          pltpu.SemaphoreType.DMA,
          pltpu.SemaphoreType.DMA,
          pltpu.SemaphoreType.DMA,
      ],
  )
  def k(table_hbm, idx_hbm, out_hbm, idx_v, buf0, buf1,
        gsem0, gsem1, ssem0, ssem1):
    wid = lax.axis_index("s") * 2 + lax.axis_index("c")
    base = wid * NB_PER_W
    pltpu.sync_copy(idx_hbm.at[pl.ds(base, NB_PER_W)], idx_v)

    bufs = (buf0, buf1)
    gsems = (gsem0, gsem1)
    ssems = (ssem0, ssem1)

    def gather(j, p):
      return pltpu.async_copy(table_hbm.at[idx_v.at[j]], bufs[p], gsems[p])

    def store(j, p):
      return pltpu.async_copy(bufs[p], out_hbm.at[base + j], ssems[p])

    gather(0, 0).wait()
    gather(1, 1)
    store(0, 0)

    def pair_body(m, carry):
      g = 1 + 2 * m
      for (j, p) in ((g, 1), (g + 1, 0)):
        q = 1 - p
        pltpu.make_async_copy(
            table_hbm.at[idx_v.at[j]], bufs[p], gsems[p]).wait()
        pltpu.make_async_copy(
            bufs[q], out_hbm.at[base + j - 1], ssems[q]).wait()
        gather(j + 1, q)
        store(j, p)
      return carry

    lax.fori_loop(0, (NB_PER_W - 2) // 2, pair_body, 0, unroll=False)

    j_last = NB_PER_W - 1
    pltpu.make_async_copy(
        table_hbm.at[idx_v.at[j_last]], bufs[1], gsems[1]).wait()
    pltpu.make_async_copy(
        bufs[0], out_hbm.at[base + j_last - 1], ssems[0]).wait()
    store(j_last, 1)
    pltpu.make_async_copy(
        bufs[1], out_hbm.at[base + j_last], ssems[1]).wait()

  return k(table, idx)


def _tc_unpad(padded):
  def body(in_ref, out_ref):
    out_ref[...] = in_ref[:, :SEQ, :]

  return pl.pallas_call(
      body,
      grid=(BATCH // TC_BB,),
      in_specs=[pl.BlockSpec((TC_BB, SEQ_PAD, EMBED_DIM), lambda i: (i, 0, 0))],
      out_specs=pl.BlockSpec((TC_BB, SEQ, EMBED_DIM), lambda i: (i, 0, 0)),
      out_shape=jax.ShapeDtypeStruct((BATCH, SEQ, EMBED_DIM), jnp.float32),
  )(padded)


def kernel(x, token_embedding):
  idx_pad = jnp.pad(x.astype(jnp.int32), ((0, 0), (0, SEQ_PAD - SEQ)),
                    mode="edge")
  padded = _sc_gather_padded(token_embedding, idx_pad)
  return padded[:, :SEQ, :]
